# baseline (speedup 1.0000x reference)
import functools

import jax
import jax.numpy as jnp
from jax import lax
from jax.experimental import pallas as pl
from jax.experimental.pallas import tpu as pltpu
from jax.experimental.pallas import tpu_sc as plsc

G = 64
H = 512
W = 512
C = 3
B = 32
GA = G + 8


NB = 4


def _start_row_copy(s_ref, x_hbm, rows_vmem, sems, step, ring):
    for j in range(NB):
        b = step * NB + j
        r0a = pl.multiple_of(s_ref[b, 2], 8)
        pltpu.make_async_copy(
            x_hbm.at[b, :, pl.ds(r0a, GA), :],
            rows_vmem.at[ring, j], sems.at[ring, j],
        ).start()


def _tc_full_body(s_ref, x_hbm, full_ref, rows_vmem, sems):
    step = pl.program_id(0)
    ring = lax.rem(step, 2)

    @pl.when(step == 0)
    def _prologue():
        _start_row_copy(s_ref, x_hbm, rows_vmem, sems, step, ring)

    @pl.when(step + 1 < B // NB)
    def _prefetch():
        _start_row_copy(s_ref, x_hbm, rows_vmem, sems, step + 1, 1 - ring)

    full_ref[...] = jnp.zeros((NB, C, H, W), jnp.float32)

    for j in range(NB):
        b = step * NB + j
        rstart = s_ref[b, 0]
        cstart = s_ref[b, 1]
        r0a = pl.multiple_of(s_ref[b, 2], 8)
        pltpu.make_async_copy(
            x_hbm.at[b, :, pl.ds(r0a, GA), :],
            rows_vmem.at[ring, j], sems.at[ring, j],
        ).wait()
        rid = r0a + lax.broadcasted_iota(jnp.int32, (C, GA, W), 1)
        cid = lax.broadcasted_iota(jnp.int32, (C, GA, W), 2)
        mask = ((rid >= rstart) & (rid < rstart + G)
                & (cid >= cstart) & (cid < cstart + G))
        full_ref[j, :, pl.ds(r0a, GA), :] = jnp.where(
            mask, rows_vmem[ring, j], 0.0)


def _tc_full(scalars, x):
    return pl.pallas_call(
        _tc_full_body,
        grid=(B // NB,),
        in_specs=[
            pl.BlockSpec(memory_space=pltpu.SMEM),
            pl.BlockSpec(memory_space=pl.ANY),
        ],
        out_specs=pl.BlockSpec((NB, C, H, W), lambda b: (b, 0, 0, 0)),
        out_shape=jax.ShapeDtypeStruct((B, C, H, W), jnp.float32),
        scratch_shapes=[
            pltpu.VMEM((2, NB, C, GA, W), jnp.float32),
            pltpu.SemaphoreType.DMA((2, NB)),
        ],
    )(scalars, x)


def _make_sc_patch():
    mesh = plsc.VectorSubcoreMesh(core_axis_name="c", subcore_axis_name="s")

    @functools.partial(
        pl.kernel,
        out_type=jax.ShapeDtypeStruct((B * C * G * G,), jnp.float32),
        mesh=mesh,
        scratch_types=[
            pltpu.VMEM((2 * B,), jnp.int32),
            pltpu.VMEM((G,), jnp.int32),
            pltpu.VMEM((G, W), jnp.float32),
            pltpu.VMEM((G * G,), jnp.float32),
            pltpu.SemaphoreType.DMA,
        ],
        compiler_params=pltpu.CompilerParams(needs_layout_passes=False),
    )
    def sc_patch(x_hbm, l_hbm, out_hbm, l_v, idx_v, staged, out_v, sem):
        cid = lax.axis_index("c")
        sid = lax.axis_index("s")
        wid = sid * 2 + cid
        pltpu.sync_copy(l_hbm, l_v)
        lane = lax.iota(jnp.int32, 16)
        for t in range(3):
            chan = wid * 3 + t
            b = chan // 3
            cx = plsc.load_gather(l_v, [jnp.full((16,), 2 * b, jnp.int32)]) - 1
            cy = plsc.load_gather(l_v, [jnp.full((16,), 2 * b + 1, jnp.int32)]) - 1
            for k in range(4):
                r = cx - (G // 2) + (lane + 16 * k)
                idx_v[pl.ds(16 * k, 16)] = jnp.clip(r, 0, H - 1) + chan * H
            pltpu.async_copy(x_hbm.at[idx_v], staged, sem).wait()
            cbase = cy - (G // 2)

            def row_body(i, carry, cx=cx, cbase=cbase):
                rv = (cx - (G // 2) + i >= 0) & (cx - (G // 2) + i < H)
                row_i = jnp.full((16,), i, jnp.int32)
                for k in range(4):
                    col = cbase + (lane + 16 * k)
                    cv = (col >= 0) & (col < W)
                    val = plsc.load_gather(
                        staged, [row_i, jnp.clip(col, 0, W - 1)])
                    val = jnp.where(rv & cv, val, 0.0)
                    out_v[pl.ds(i * G + 16 * k, 16)] = val
                return carry

            lax.fori_loop(0, G, row_body, 0)
            pltpu.sync_copy(out_v, out_hbm.at[pl.ds(chan * G * G, G * G)])

    return sc_patch


_sc_patch_cache = []


def _sc_patch(x2d, lflat):
    if not _sc_patch_cache:
        _sc_patch_cache.append(_make_sc_patch())
    return _sc_patch_cache[0](x2d, lflat)


def kernel(x, l):
    coords = l.astype(jnp.int32) - 1
    rstart = coords[:, 0] - G // 2
    cstart = coords[:, 1] - G // 2
    r0 = jnp.clip(rstart, 0, H - G)
    r0a = jnp.minimum(r0 & ~7, H - GA)
    scalars = jnp.stack([rstart, cstart, r0a], axis=-1)
    full = _tc_full(scalars, x)
    x2d = x.reshape(B * C * H, W)
    lflat = l.astype(jnp.int32).reshape(2 * B)
    patch = _sc_patch(x2d, lflat).reshape(B, C, G, G)
    return full, patch


# speedup vs baseline: 1.0602x; 1.0602x over previous
import functools

import jax
import jax.numpy as jnp
from jax import lax
from jax.experimental import pallas as pl
from jax.experimental.pallas import tpu as pltpu
from jax.experimental.pallas import tpu_sc as plsc

G = 64
H = 512
W = 512
C = 3
B = 32
GA = G + 8


NB = 4


def _start_row_copy(s_ref, x_hbm, rows_vmem, sems, step, ring):
    for j in range(NB):
        b = step * NB + j
        r0a = pl.multiple_of(s_ref[b, 2], 8)
        pltpu.make_async_copy(
            x_hbm.at[b, :, pl.ds(r0a, GA), :],
            rows_vmem.at[ring, j], sems.at[ring, j],
        ).start()


def _tc_full_body(s_ref, x_hbm, full_ref, rows_vmem, sems):
    step = pl.program_id(0)
    ring = lax.rem(step, 2)

    @pl.when(step == 0)
    def _prologue():
        _start_row_copy(s_ref, x_hbm, rows_vmem, sems, step, ring)

    @pl.when(step + 1 < B // NB)
    def _prefetch():
        _start_row_copy(s_ref, x_hbm, rows_vmem, sems, step + 1, 1 - ring)

    full_ref[...] = jnp.zeros((NB, C, H, W), jnp.float32)

    for j in range(NB):
        b = step * NB + j
        rstart = s_ref[b, 0]
        cstart = s_ref[b, 1]
        r0a = pl.multiple_of(s_ref[b, 2], 8)
        pltpu.make_async_copy(
            x_hbm.at[b, :, pl.ds(r0a, GA), :],
            rows_vmem.at[ring, j], sems.at[ring, j],
        ).wait()
        rid = r0a + lax.broadcasted_iota(jnp.int32, (C, GA, W), 1)
        cid = lax.broadcasted_iota(jnp.int32, (C, GA, W), 2)
        mask = ((rid >= rstart) & (rid < rstart + G)
                & (cid >= cstart) & (cid < cstart + G))
        full_ref[j, :, pl.ds(r0a, GA), :] = jnp.where(
            mask, rows_vmem[ring, j], 0.0)


def _tc_full(scalars, x):
    return pl.pallas_call(
        _tc_full_body,
        grid=(B // NB,),
        in_specs=[
            pl.BlockSpec(memory_space=pltpu.SMEM),
            pl.BlockSpec(memory_space=pl.ANY),
        ],
        out_specs=pl.BlockSpec((NB, C, H, W), lambda b: (b, 0, 0, 0)),
        out_shape=jax.ShapeDtypeStruct((B, C, H, W), jnp.float32),
        scratch_shapes=[
            pltpu.VMEM((2, NB, C, GA, W), jnp.float32),
            pltpu.SemaphoreType.DMA((2, NB)),
        ],
    )(scalars, x)


def _make_sc_patch():
    mesh = plsc.VectorSubcoreMesh(core_axis_name="c", subcore_axis_name="s")

    @functools.partial(
        pl.kernel,
        out_type=jax.ShapeDtypeStruct((B * C * G * G,), jnp.float32),
        mesh=mesh,
        scratch_types=[
            pltpu.VMEM((2 * B,), jnp.int32),
            pltpu.VMEM((G,), jnp.int32),
            pltpu.VMEM((G,), jnp.int32),
            pltpu.VMEM((G,), jnp.int32),
            pltpu.VMEM((3, G, W), jnp.float32),
            pltpu.VMEM((G * G,), jnp.float32),
            pltpu.VMEM((G * G,), jnp.float32),
            pltpu.VMEM((G * G,), jnp.float32),
            pltpu.SemaphoreType.DMA((3,)),
            pltpu.SemaphoreType.DMA((3,)),
        ],
        compiler_params=pltpu.CompilerParams(needs_layout_passes=False),
    )
    def sc_patch(x_hbm, l_hbm, out_hbm, l_v, idx0, idx1, idx2, staged,
                 out0, out1, out2, gsem, osem):
        idx_refs = [idx0, idx1, idx2]
        out_refs = [out0, out1, out2]
        cid = lax.axis_index("c")
        sid = lax.axis_index("s")
        wid = sid * 2 + cid
        pltpu.sync_copy(l_hbm, l_v)
        lane = lax.iota(jnp.int32, 16)
        cxs, cys, gathers = [], [], []
        for t in range(3):
            chan = wid * 3 + t
            b = chan // 3
            cx = plsc.load_gather(l_v, [jnp.full((16,), 2 * b, jnp.int32)]) - 1
            cy = plsc.load_gather(l_v, [jnp.full((16,), 2 * b + 1, jnp.int32)]) - 1
            for k in range(4):
                r = cx - (G // 2) + (lane + 16 * k)
                idx_refs[t][pl.ds(16 * k, 16)] = (
                    jnp.clip(r, 0, H - 1) + chan * H)
            gathers.append(
                pltpu.async_copy(x_hbm.at[idx_refs[t]], staged.at[t],
                                 gsem.at[t]))
            cxs.append(cx)
            cys.append(cy)
        outs = []
        for t in range(3):
            chan = wid * 3 + t
            gathers[t].wait()
            cx, cbase = cxs[t], cys[t] - (G // 2)

            def row_body(i, carry, cx=cx, cbase=cbase, t=t):
                rv = (cx - (G // 2) + i >= 0) & (cx - (G // 2) + i < H)
                row_i = jnp.full((16,), i, jnp.int32)
                for k in range(4):
                    col = cbase + (lane + 16 * k)
                    cv = (col >= 0) & (col < W)
                    val = plsc.load_gather(
                        staged.at[t], [row_i, jnp.clip(col, 0, W - 1)])
                    val = jnp.where(rv & cv, val, 0.0)
                    out_refs[t][pl.ds(i * G + 16 * k, 16)] = val
                return carry

            lax.fori_loop(0, G, row_body, 0)
            outs.append(
                pltpu.async_copy(out_refs[t],
                                 out_hbm.at[pl.ds(chan * G * G, G * G)],
                                 osem.at[t]))
        for t in range(3):
            outs[t].wait()

    return sc_patch


_sc_patch_cache = []


def _sc_patch(x2d, lflat):
    if not _sc_patch_cache:
        _sc_patch_cache.append(_make_sc_patch())
    return _sc_patch_cache[0](x2d, lflat)


def kernel(x, l):
    coords = l.astype(jnp.int32) - 1
    rstart = coords[:, 0] - G // 2
    cstart = coords[:, 1] - G // 2
    r0 = jnp.clip(rstart, 0, H - G)
    r0a = jnp.minimum(r0 & ~7, H - GA)
    scalars = jnp.stack([rstart, cstart, r0a], axis=-1)
    full = _tc_full(scalars, x)
    x2d = x.reshape(B * C * H, W)
    lflat = l.astype(jnp.int32).reshape(2 * B)
    patch = _sc_patch(x2d, lflat).reshape(B, C, G, G)
    return full, patch


# speedup vs baseline: 1.1067x; 1.0438x over previous
import functools

import jax
import jax.numpy as jnp
from jax import lax
from jax.experimental import pallas as pl
from jax.experimental.pallas import tpu as pltpu
from jax.experimental.pallas import tpu_sc as plsc

G = 64
H = 512
W = 512
C = 3
B = 32
GA = G + 8


NB = 4


def _start_row_copy(s_ref, x_hbm, rows_vmem, sems, step, ring):
    for j in range(NB):
        b = step * NB + j
        r0a = pl.multiple_of(s_ref[b, 2], 8)
        pltpu.make_async_copy(
            x_hbm.at[b, :, pl.ds(r0a, GA), :],
            rows_vmem.at[ring, j], sems.at[ring, j],
        ).start()


def _tc_full_body(s_ref, x_hbm, full_ref, rows_vmem, sems):
    step = pl.program_id(0)
    ring = lax.rem(step, 2)

    @pl.when(step == 0)
    def _prologue():
        _start_row_copy(s_ref, x_hbm, rows_vmem, sems, step, ring)

    @pl.when(step + 1 < B // NB)
    def _prefetch():
        _start_row_copy(s_ref, x_hbm, rows_vmem, sems, step + 1, 1 - ring)

    full_ref[...] = jnp.zeros((NB, C, H, W), jnp.float32)

    for j in range(NB):
        b = step * NB + j
        rstart = s_ref[b, 0]
        cstart = s_ref[b, 1]
        r0a = pl.multiple_of(s_ref[b, 2], 8)
        pltpu.make_async_copy(
            x_hbm.at[b, :, pl.ds(r0a, GA), :],
            rows_vmem.at[ring, j], sems.at[ring, j],
        ).wait()
        rid = r0a + lax.broadcasted_iota(jnp.int32, (C, GA, W), 1)
        cid = lax.broadcasted_iota(jnp.int32, (C, GA, W), 2)
        mask = ((rid >= rstart) & (rid < rstart + G)
                & (cid >= cstart) & (cid < cstart + G))
        full_ref[j, :, pl.ds(r0a, GA), :] = jnp.where(
            mask, rows_vmem[ring, j], 0.0)


def _tc_full(scalars, x):
    return pl.pallas_call(
        _tc_full_body,
        grid=(B // NB,),
        in_specs=[
            pl.BlockSpec(memory_space=pltpu.SMEM),
            pl.BlockSpec(memory_space=pl.ANY),
        ],
        out_specs=pl.BlockSpec((NB, C, H, W), lambda b: (b, 0, 0, 0)),
        out_shape=jax.ShapeDtypeStruct((B, C, H, W), jnp.float32),
        scratch_shapes=[
            pltpu.VMEM((2, NB, C, GA, W), jnp.float32),
            pltpu.SemaphoreType.DMA((2, NB)),
        ],
    )(scalars, x)


def _make_sc_patch():
    mesh = plsc.VectorSubcoreMesh(core_axis_name="c", subcore_axis_name="s")

    @functools.partial(
        pl.kernel,
        out_type=jax.ShapeDtypeStruct((B, C, G, G), jnp.float32),
        mesh=mesh,
        scratch_types=[
            pltpu.VMEM((2 * B,), jnp.int32),
            pltpu.VMEM((G,), jnp.int32),
            pltpu.VMEM((G,), jnp.int32),
            pltpu.VMEM((G,), jnp.int32),
            pltpu.VMEM((3, G, W), jnp.float32),
            pltpu.VMEM((G, G), jnp.float32),
            pltpu.VMEM((G, G), jnp.float32),
            pltpu.VMEM((G, G), jnp.float32),
            pltpu.SemaphoreType.DMA((3,)),
            pltpu.SemaphoreType.DMA((3,)),
        ],
        compiler_params=pltpu.CompilerParams(needs_layout_passes=False),
    )
    def sc_patch(x_hbm, l_hbm, out_hbm, l_v, idx0, idx1, idx2, staged,
                 out0, out1, out2, gsem, osem):
        idx_refs = [idx0, idx1, idx2]
        out_refs = [out0, out1, out2]
        cid = lax.axis_index("c")
        sid = lax.axis_index("s")
        wid = sid * 2 + cid
        pltpu.sync_copy(l_hbm, l_v)
        lane = lax.iota(jnp.int32, 16)
        cxs, cys, gathers = [], [], []
        for t in range(3):
            chan = wid * 3 + t
            b = chan // 3
            cx = plsc.load_gather(l_v, [jnp.full((16,), 2 * b, jnp.int32)]) - 1
            cy = plsc.load_gather(l_v, [jnp.full((16,), 2 * b + 1, jnp.int32)]) - 1
            for k in range(4):
                r = cx - (G // 2) + (lane + 16 * k)
                idx_refs[t][pl.ds(16 * k, 16)] = (
                    jnp.clip(r, 0, H - 1) + chan * H)
            gathers.append(
                pltpu.async_copy(x_hbm.at[idx_refs[t]], staged.at[t],
                                 gsem.at[t]))
            cxs.append(cx)
            cys.append(cy)
        outs = []
        for t in range(3):
            chan = wid * 3 + t
            gathers[t].wait()
            cx, cbase = cxs[t], cys[t] - (G // 2)

            def row_body(i, carry, cx=cx, cbase=cbase, t=t):
                rv = (cx - (G // 2) + i >= 0) & (cx - (G // 2) + i < H)
                row_i = jnp.full((16,), i, jnp.int32)
                for k in range(4):
                    col = cbase + (lane + 16 * k)
                    cv = (col >= 0) & (col < W)
                    val = plsc.load_gather(
                        staged.at[t], [row_i, jnp.clip(col, 0, W - 1)])
                    val = jnp.where(rv & cv, val, 0.0)
                    out_refs[t][i, pl.ds(16 * k, 16)] = val
                return carry

            lax.fori_loop(0, G, row_body, 0)
            b2 = chan // 3
            c2 = chan - b2 * 3
            outs.append(
                pltpu.async_copy(out_refs[t], out_hbm.at[b2, c2],
                                 osem.at[t]))
        for t in range(3):
            outs[t].wait()

    return sc_patch


_sc_patch_cache = []


def _sc_patch(x2d, lflat):
    if not _sc_patch_cache:
        _sc_patch_cache.append(_make_sc_patch())
    return _sc_patch_cache[0](x2d, lflat)


def kernel(x, l):
    coords = l.astype(jnp.int32) - 1
    rstart = coords[:, 0] - G // 2
    cstart = coords[:, 1] - G // 2
    r0 = jnp.clip(rstart, 0, H - G)
    r0a = jnp.minimum(r0 & ~7, H - GA)
    scalars = jnp.stack([rstart, cstart, r0a], axis=-1)
    full = _tc_full(scalars, x)
    x2d = x.reshape(B * C * H, W)
    lflat = l.astype(jnp.int32).reshape(2 * B)
    patch = _sc_patch(x2d, lflat)
    return full, patch


# speedup vs baseline: 1.1462x; 1.0357x over previous
import functools

import jax
import jax.numpy as jnp
from jax import lax
from jax.experimental import pallas as pl
from jax.experimental.pallas import tpu as pltpu
from jax.experimental.pallas import tpu_sc as plsc

G = 64
H = 512
W = 512
C = 3
B = 32
GA = G + 8
WA = 256


NB = 4


def _start_row_copy(s_ref, x_hbm, rows_vmem, sems, step, ring):
    for j in range(NB):
        b = step * NB + j
        r0a = pl.multiple_of(s_ref[b, 2], 8)
        c0a = pl.multiple_of(s_ref[b, 3], 128)
        pltpu.make_async_copy(
            x_hbm.at[b, :, pl.ds(r0a, GA), pl.ds(c0a, WA)],
            rows_vmem.at[ring, j], sems.at[ring, j],
        ).start()


def _tc_full_body(s_ref, x_hbm, full_ref, rows_vmem, sems):
    step = pl.program_id(0)
    ring = lax.rem(step, 2)

    @pl.when(step == 0)
    def _prologue():
        _start_row_copy(s_ref, x_hbm, rows_vmem, sems, step, ring)

    @pl.when(step + 1 < B // NB)
    def _prefetch():
        _start_row_copy(s_ref, x_hbm, rows_vmem, sems, step + 1, 1 - ring)

    full_ref[...] = jnp.zeros((NB, C, H, W), jnp.float32)

    for j in range(NB):
        b = step * NB + j
        rstart = s_ref[b, 0]
        cstart = s_ref[b, 1]
        r0a = pl.multiple_of(s_ref[b, 2], 8)
        c0a = pl.multiple_of(s_ref[b, 3], 128)
        pltpu.make_async_copy(
            x_hbm.at[b, :, pl.ds(r0a, GA), pl.ds(c0a, WA)],
            rows_vmem.at[ring, j], sems.at[ring, j],
        ).wait()
        rid = r0a + lax.broadcasted_iota(jnp.int32, (C, GA, WA), 1)
        cid = c0a + lax.broadcasted_iota(jnp.int32, (C, GA, WA), 2)
        mask = ((rid >= rstart) & (rid < rstart + G)
                & (cid >= cstart) & (cid < cstart + G))
        full_ref[j, :, pl.ds(r0a, GA), pl.ds(c0a, WA)] = jnp.where(
            mask, rows_vmem[ring, j], 0.0)


def _tc_full(scalars, x):
    return pl.pallas_call(
        _tc_full_body,
        grid=(B // NB,),
        in_specs=[
            pl.BlockSpec(memory_space=pltpu.SMEM),
            pl.BlockSpec(memory_space=pl.ANY),
        ],
        out_specs=pl.BlockSpec((NB, C, H, W), lambda b: (b, 0, 0, 0)),
        out_shape=jax.ShapeDtypeStruct((B, C, H, W), jnp.float32),
        scratch_shapes=[
            pltpu.VMEM((2, NB, C, GA, WA), jnp.float32),
            pltpu.SemaphoreType.DMA((2, NB)),
        ],
    )(scalars, x)


def _make_sc_patch():
    mesh = plsc.VectorSubcoreMesh(core_axis_name="c", subcore_axis_name="s")

    @functools.partial(
        pl.kernel,
        out_type=jax.ShapeDtypeStruct((B, C, G, G), jnp.float32),
        mesh=mesh,
        scratch_types=[
            pltpu.VMEM((2 * B,), jnp.int32),
            pltpu.VMEM((G,), jnp.int32),
            pltpu.VMEM((G,), jnp.int32),
            pltpu.VMEM((G,), jnp.int32),
            pltpu.VMEM((3, G, W), jnp.float32),
            pltpu.VMEM((G, G), jnp.float32),
            pltpu.VMEM((G, G), jnp.float32),
            pltpu.VMEM((G, G), jnp.float32),
            pltpu.SemaphoreType.DMA((3,)),
            pltpu.SemaphoreType.DMA((3,)),
        ],
        compiler_params=pltpu.CompilerParams(needs_layout_passes=False),
    )
    def sc_patch(x_hbm, l_hbm, out_hbm, l_v, idx0, idx1, idx2, staged,
                 out0, out1, out2, gsem, osem):
        idx_refs = [idx0, idx1, idx2]
        out_refs = [out0, out1, out2]
        cid = lax.axis_index("c")
        sid = lax.axis_index("s")
        wid = sid * 2 + cid
        pltpu.sync_copy(l_hbm, l_v)
        lane = lax.iota(jnp.int32, 16)
        cxs, cys, gathers = [], [], []
        for t in range(3):
            chan = wid * 3 + t
            b = chan // 3
            cx = plsc.load_gather(l_v, [jnp.full((16,), 2 * b, jnp.int32)]) - 1
            cy = plsc.load_gather(l_v, [jnp.full((16,), 2 * b + 1, jnp.int32)]) - 1
            for k in range(4):
                r = cx - (G // 2) + (lane + 16 * k)
                idx_refs[t][pl.ds(16 * k, 16)] = (
                    jnp.clip(r, 0, H - 1) + chan * H)
            gathers.append(
                pltpu.async_copy(x_hbm.at[idx_refs[t]], staged.at[t],
                                 gsem.at[t]))
            cxs.append(cx)
            cys.append(cy)
        outs = []
        for t in range(3):
            chan = wid * 3 + t
            gathers[t].wait()
            cx, cbase = cxs[t], cys[t] - (G // 2)

            def row_body(i, carry, cx=cx, cbase=cbase, t=t):
                rv = (cx - (G // 2) + i >= 0) & (cx - (G // 2) + i < H)
                row_i = jnp.full((16,), i, jnp.int32)
                for k in range(4):
                    col = cbase + (lane + 16 * k)
                    cv = (col >= 0) & (col < W)
                    val = plsc.load_gather(
                        staged.at[t], [row_i, jnp.clip(col, 0, W - 1)])
                    val = jnp.where(rv & cv, val, 0.0)
                    out_refs[t][i, pl.ds(16 * k, 16)] = val
                return carry

            lax.fori_loop(0, G, row_body, 0)
            b2 = chan // 3
            c2 = chan - b2 * 3
            outs.append(
                pltpu.async_copy(out_refs[t], out_hbm.at[b2, c2],
                                 osem.at[t]))
        for t in range(3):
            outs[t].wait()

    return sc_patch


_sc_patch_cache = []


def _sc_patch(x2d, lflat):
    if not _sc_patch_cache:
        _sc_patch_cache.append(_make_sc_patch())
    return _sc_patch_cache[0](x2d, lflat)


def kernel(x, l):
    coords = l.astype(jnp.int32) - 1
    rstart = coords[:, 0] - G // 2
    cstart = coords[:, 1] - G // 2
    r0 = jnp.clip(rstart, 0, H - G)
    r0a = jnp.minimum(r0 & ~7, H - GA)
    c0a = jnp.minimum(jnp.maximum(cstart, 0) & ~127, W - WA)
    scalars = jnp.stack([rstart, cstart, r0a, c0a], axis=-1)
    full = _tc_full(scalars, x)
    x2d = x.reshape(B * C * H, W)
    lflat = l.astype(jnp.int32).reshape(2 * B)
    patch = _sc_patch(x2d, lflat)
    return full, patch
